# baseline (speedup 1.0000x reference)
import functools

import jax
import jax.numpy as jnp
from jax import lax
from jax.experimental import pallas as pl
from jax.experimental.pallas import tpu as pltpu
from jax.experimental.pallas import tpu_sc as plsc

B, T, D = 4, 8192, 768
P, LMAX, C = 4096, 8, 64
CAT = 2 * C
NC, NS = 2, 16
NW = NC * NS
CHUNK = 16
RPC = CHUNK * LMAX
NJ = C // 16
BM = 2048
NB0 = 2

_NEG = -1e30


def _mm_body(x_ref, w_ref, o_ref):
    i = pl.program_id(0)
    n = pl.num_programs(0)

    @pl.when(i < n - 1)
    def _():
        o_ref[...] = jnp.dot(x_ref[...], w_ref[...],
                             preferred_element_type=jnp.float32)

    @pl.when(i == n - 1)
    def _():
        col = lax.broadcasted_iota(jnp.int32, (BM, CAT), 1)
        o_ref[...] = jnp.where(col < C, 0.0, _NEG)


def _project_rows(x, w_cat, row0, nrows):
    blk0 = row0 // BM
    nblk = nrows // BM
    return pl.pallas_call(
        _mm_body,
        grid=(nblk + 1,),
        in_specs=[
            pl.BlockSpec((BM, D),
                         lambda i: (jnp.minimum(blk0 + i, blk0 + nblk - 1),
                                    0)),
            pl.BlockSpec((D, CAT), lambda i: (0, 0)),
        ],
        out_specs=pl.BlockSpec((BM, CAT), lambda i: (i, 0)),
        out_shape=jax.ShapeDtypeStruct((nrows + BM, CAT), jnp.float32),
    )(x, w_cat)


def _tree_sum(vals):
    vals = list(vals)
    while len(vals) > 1:
        vals = [vals[i] + vals[i + 1] for i in range(0, len(vals) - 1, 2)] \
            + ([vals[-1]] if len(vals) % 2 else [])
    return vals[0]


_mesh = plsc.VectorSubcoreMesh(core_axis_name="c", subcore_axis_name="s")


def _make_sc_pool(nphrase, nprev):
    ppw = nphrase // NW
    nchunk = ppw // CHUNK
    prev_pw = nprev // NW if nprev else 0

    scratch = [
        pltpu.VMEM((ppw * LMAX,), jnp.int32),
        pltpu.VMEM((LMAX * C,), jnp.float32),
        pltpu.VMEM((2, RPC, CAT), jnp.float32),
        pltpu.VMEM((2, CHUNK, C), jnp.float32),
        pltpu.SemaphoreType.DMA,
        pltpu.SemaphoreType.DMA,
        pltpu.SemaphoreType.DMA,
        pltpu.SemaphoreType.DMA,
    ]
    if nprev:
        scratch.append(pltpu.VMEM((prev_pw, C), jnp.float32))
        scratch.append(pltpu.SemaphoreType.DMA)

    @functools.partial(
        pl.kernel,
        mesh=_mesh,
        out_type=jax.ShapeDtypeStruct((nprev + nphrase, C), jnp.float32),
        scratch_types=scratch,
    )
    def _sc_pool(*refs):
        if nprev:
            (cat_hbm, idx_hbm, bpos_hbm, prev_hbm, out_hbm,
             idx_v, bpos_v, rows_v, out_v,
             gsem0, gsem1, osem0, osem1, prev_v, psem) = refs
        else:
            (cat_hbm, idx_hbm, bpos_hbm, out_hbm,
             idx_v, bpos_v, rows_v, out_v,
             gsem0, gsem1, osem0, osem1) = refs
        wid = lax.axis_index("s") * NC + lax.axis_index("c")
        start_w = wid * ppw
        ostart_w = nprev + start_w
        gsem = (gsem0, gsem1)
        osem = (osem0, osem1)

        pltpu.sync_copy(idx_hbm.at[pl.ds(start_w * LMAX, ppw * LMAX)], idx_v)

        def _gather(ci, buf):
            idx_slice = idx_v.at[pl.ds(ci * RPC, RPC)]
            return pltpu.async_copy(cat_hbm.at[idx_slice], rows_v.at[buf],
                                    gsem[buf])

        _gather(0, 0)
        pltpu.sync_copy(bpos_hbm, bpos_v)
        if nprev:
            pcopy = pltpu.async_copy(
                prev_hbm.at[pl.ds(wid * prev_pw, prev_pw)], prev_v, psem)

        bpos = [[bpos_v[pl.ds(l * C + 16 * j, 16)] for j in range(NJ)]
                for l in range(LMAX)]

        def pair_body(g, carry):
            for bu in range(2):
                ci = 2 * g + bu
                nci = jnp.minimum(ci + 1, nchunk - 1)
                _gather(nci, 1 - bu)
                pltpu.make_async_copy(
                    cat_hbm.at[idx_v.at[pl.ds(ci * RPC, RPC)]],
                    rows_v.at[bu], gsem[bu]).wait()
                ob = bu

                @pl.when(ci >= 2)
                def _():
                    pltpu.make_async_copy(
                        out_v.at[ob],
                        out_hbm.at[pl.ds(ostart_w + (ci - 2) * CHUNK,
                                         CHUNK)],
                        osem[ob]).wait()

                @plsc.parallel_loop(0, CHUNK, unroll=8)
                def phrase_body(p):
                    base = p * LMAX
                    for j in range(NJ):
                        e = [jnp.exp(rows_v[bu, base + l,
                                            pl.ds(C + 16 * j, 16)]
                                     + bpos[l][j])
                             for l in range(LMAX)]
                        s = _tree_sum(e)
                        prods = [e[l] * rows_v[bu, base + l,
                                               pl.ds(16 * j, 16)]
                                 for l in range(LMAX)]
                        out_v[ob, p, pl.ds(16 * j, 16)] = _tree_sum(prods) / s

                pltpu.async_copy(
                    out_v.at[ob],
                    out_hbm.at[pl.ds(ostart_w + ci * CHUNK, CHUNK)],
                    osem[ob])
            return carry

        lax.fori_loop(0, nchunk // 2, pair_body, 0)

        if nprev:
            pcopy.wait()
            pltpu.sync_copy(prev_v, out_hbm.at[pl.ds(wid * prev_pw,
                                                     prev_pw)])
        pltpu.make_async_copy(
            cat_hbm.at[idx_v.at[pl.ds((nchunk - 1) * RPC, RPC)]],
            rows_v.at[0], gsem[0]).wait()
        for bu in range(2):
            ci = nchunk - 2 + bu
            pltpu.make_async_copy(
                out_v.at[bu],
                out_hbm.at[pl.ds(ostart_w + ci * CHUNK, CHUNK)],
                osem[bu]).wait()

    return _sc_pool


_sc_pool_first = _make_sc_pool(NB0 * P, 0)
_sc_pool_last = _make_sc_pool((B - NB0) * P, NB0 * P)


def kernel(h, phrase_mask, phrase_token_idx, W_kv, W_z, B_pos):
    w_cat = jnp.concatenate([W_kv, W_z], axis=1)
    bpos_flat = B_pos.astype(jnp.float32).reshape(-1)

    x = h.reshape(B * T, D)
    out = None
    for stage, (b0, nb) in enumerate(((0, NB0), (NB0, B - NB0))):
        sl = slice(b0, b0 + nb)
        cat = _project_rows(x, w_cat, b0 * T, nb * T)
        tok = phrase_token_idx[sl].astype(jnp.int32)
        boff = (jnp.arange(nb, dtype=jnp.int32) * T)[:, None, None]
        idx_h = jnp.where(phrase_mask[sl], tok + boff,
                          nb * T + (tok & (BM - 1))).reshape(-1)
        if stage == 0:
            out = _sc_pool_first(cat, idx_h, bpos_flat)
        else:
            out = _sc_pool_last(cat, idx_h, bpos_flat, out)
    return out.reshape(B, P, C)


# speedup vs baseline: 1.5810x; 1.5810x over previous
import functools

import jax
import jax.numpy as jnp
from jax import lax
from jax.experimental import pallas as pl
from jax.experimental.pallas import tpu as pltpu
from jax.experimental.pallas import tpu_sc as plsc

B, T, D = 4, 8192, 768
P, LMAX, C = 4096, 8, 64
CAT = 2 * C
NC, NS = 2, 16
NW = NC * NS
CHUNK = 8
RPC = CHUNK * LMAX
NJ = C // 16
BM = 2048
NB0 = 2

_NEG = -1e30


def _mm_body(x_ref, w_ref, o_ref):
    i = pl.program_id(0)
    n = pl.num_programs(0)

    @pl.when(i < n - 1)
    def _():
        o_ref[...] = jnp.dot(x_ref[...], w_ref[...],
                             preferred_element_type=jnp.float32)

    @pl.when(i == n - 1)
    def _():
        col = lax.broadcasted_iota(jnp.int32, (BM, CAT), 1)
        o_ref[...] = jnp.where(col < C, 0.0, _NEG)


def _project_rows(x, w_cat, row0, nrows):
    blk0 = row0 // BM
    nblk = nrows // BM
    return pl.pallas_call(
        _mm_body,
        grid=(nblk + 1,),
        in_specs=[
            pl.BlockSpec((BM, D),
                         lambda i: (jnp.minimum(blk0 + i, blk0 + nblk - 1),
                                    0)),
            pl.BlockSpec((D, CAT), lambda i: (0, 0)),
        ],
        out_specs=pl.BlockSpec((BM, CAT), lambda i: (i, 0)),
        out_shape=jax.ShapeDtypeStruct((nrows + BM, CAT), jnp.float32),
    )(x, w_cat)


def _tree_sum(vals):
    vals = list(vals)
    while len(vals) > 1:
        vals = [vals[i] + vals[i + 1] for i in range(0, len(vals) - 1, 2)] \
            + ([vals[-1]] if len(vals) % 2 else [])
    return vals[0]


_mesh = plsc.VectorSubcoreMesh(core_axis_name="c", subcore_axis_name="s")


def _make_sc_pool(nphrase, nprev):
    ppw = nphrase // NW
    nchunk = ppw // CHUNK
    prev_pw = nprev // NW if nprev else 0

    scratch = [
        pltpu.VMEM((ppw * LMAX,), jnp.int32),
        pltpu.VMEM((LMAX * C,), jnp.float32),
        pltpu.VMEM((2, RPC, CAT), jnp.float32),
        pltpu.VMEM((2, CHUNK, C), jnp.float32),
        pltpu.SemaphoreType.DMA,
        pltpu.SemaphoreType.DMA,
        pltpu.SemaphoreType.DMA,
        pltpu.SemaphoreType.DMA,
    ]
    if nprev:
        scratch.append(pltpu.VMEM((prev_pw, C), jnp.float32))
        scratch.append(pltpu.SemaphoreType.DMA)

    @functools.partial(
        pl.kernel,
        mesh=_mesh,
        out_type=jax.ShapeDtypeStruct((nprev + nphrase, C), jnp.float32),
        scratch_types=scratch,
    )
    def _sc_pool(*refs):
        if nprev:
            (cat_hbm, idx_hbm, bpos_hbm, prev_hbm, out_hbm,
             idx_v, bpos_v, rows_v, out_v,
             gsem0, gsem1, osem0, osem1, prev_v, psem) = refs
        else:
            (cat_hbm, idx_hbm, bpos_hbm, out_hbm,
             idx_v, bpos_v, rows_v, out_v,
             gsem0, gsem1, osem0, osem1) = refs
        wid = lax.axis_index("s") * NC + lax.axis_index("c")
        start_w = wid * ppw
        ostart_w = nprev + start_w
        gsem = (gsem0, gsem1)
        osem = (osem0, osem1)

        pltpu.sync_copy(idx_hbm.at[pl.ds(start_w * LMAX, ppw * LMAX)], idx_v)

        def _gather(ci, buf):
            idx_slice = idx_v.at[pl.ds(ci * RPC, RPC)]
            return pltpu.async_copy(cat_hbm.at[idx_slice], rows_v.at[buf],
                                    gsem[buf])

        _gather(0, 0)
        pltpu.sync_copy(bpos_hbm, bpos_v)
        if nprev:
            pcopy = pltpu.async_copy(
                prev_hbm.at[pl.ds(wid * prev_pw, prev_pw)], prev_v, psem)

        bpos = [[bpos_v[pl.ds(l * C + 16 * j, 16)] for j in range(NJ)]
                for l in range(LMAX)]

        def pair_body(g, carry):
            for bu in range(2):
                ci = 2 * g + bu
                nci = jnp.minimum(ci + 1, nchunk - 1)
                _gather(nci, 1 - bu)
                pltpu.make_async_copy(
                    cat_hbm.at[idx_v.at[pl.ds(ci * RPC, RPC)]],
                    rows_v.at[bu], gsem[bu]).wait()
                ob = bu

                @pl.when(ci >= 2)
                def _():
                    pltpu.make_async_copy(
                        out_v.at[ob],
                        out_hbm.at[pl.ds(ostart_w + (ci - 2) * CHUNK,
                                         CHUNK)],
                        osem[ob]).wait()

                @plsc.parallel_loop(0, CHUNK, unroll=4)
                def phrase_body(p):
                    base = p * LMAX
                    for j in range(NJ):
                        e = [jnp.exp(rows_v[bu, base + l,
                                            pl.ds(C + 16 * j, 16)]
                                     + bpos[l][j])
                             for l in range(LMAX)]
                        s = _tree_sum(e)
                        prods = [e[l] * rows_v[bu, base + l,
                                               pl.ds(16 * j, 16)]
                                 for l in range(LMAX)]
                        out_v[ob, p, pl.ds(16 * j, 16)] = _tree_sum(prods) / s

                pltpu.async_copy(
                    out_v.at[ob],
                    out_hbm.at[pl.ds(ostart_w + ci * CHUNK, CHUNK)],
                    osem[ob])
            return carry

        lax.fori_loop(0, nchunk // 2, pair_body, 0)

        if nprev:
            pcopy.wait()
            pltpu.sync_copy(prev_v, out_hbm.at[pl.ds(wid * prev_pw,
                                                     prev_pw)])
        pltpu.make_async_copy(
            cat_hbm.at[idx_v.at[pl.ds((nchunk - 1) * RPC, RPC)]],
            rows_v.at[0], gsem[0]).wait()
        for bu in range(2):
            ci = nchunk - 2 + bu
            pltpu.make_async_copy(
                out_v.at[bu],
                out_hbm.at[pl.ds(ostart_w + ci * CHUNK, CHUNK)],
                osem[bu]).wait()

    return _sc_pool


_sc_pool_first = _make_sc_pool(NB0 * P, 0)
_sc_pool_last = _make_sc_pool((B - NB0) * P, NB0 * P)


def kernel(h, phrase_mask, phrase_token_idx, W_kv, W_z, B_pos):
    w_cat = jnp.concatenate([W_kv, W_z], axis=1)
    bpos_flat = B_pos.astype(jnp.float32).reshape(-1)

    x = h.reshape(B * T, D)
    out = None
    for stage, (b0, nb) in enumerate(((0, NB0), (NB0, B - NB0))):
        sl = slice(b0, b0 + nb)
        cat = _project_rows(x, w_cat, b0 * T, nb * T)
        tok = phrase_token_idx[sl].astype(jnp.int32)
        boff = (jnp.arange(nb, dtype=jnp.int32) * T)[:, None, None]
        idx_h = jnp.where(phrase_mask[sl], tok + boff,
                          nb * T + (tok & (BM - 1))).reshape(-1)
        if stage == 0:
            out = _sc_pool_first(cat, idx_h, bpos_flat)
        else:
            out = _sc_pool_last(cat, idx_h, bpos_flat, out)
    return out.reshape(B, P, C)


# speedup vs baseline: 1.7128x; 1.0833x over previous
import functools

import jax
import jax.numpy as jnp
from jax import lax
from jax.experimental import pallas as pl
from jax.experimental.pallas import tpu as pltpu
from jax.experimental.pallas import tpu_sc as plsc

B, T, D = 4, 8192, 768
P, LMAX, C = 4096, 8, 64
CAT = 2 * C
NC, NS = 2, 16
NW = NC * NS
CHUNK = 16
RPC = CHUNK * LMAX
NJ = C // 16
BM = 2048
NB0 = 2

_NEG = -1e30


def _mm_body(x_ref, w_ref, o_ref):
    i = pl.program_id(0)
    n = pl.num_programs(0)

    @pl.when(i < n - 1)
    def _():
        o_ref[...] = jnp.dot(x_ref[...], w_ref[...],
                             preferred_element_type=jnp.float32)

    @pl.when(i == n - 1)
    def _():
        col = lax.broadcasted_iota(jnp.int32, (BM, CAT), 1)
        o_ref[...] = jnp.where(col < C, 0.0, _NEG)


def _project_rows(x, w_cat, row0, nrows):
    blk0 = row0 // BM
    nblk = nrows // BM
    return pl.pallas_call(
        _mm_body,
        grid=(nblk + 1,),
        in_specs=[
            pl.BlockSpec((BM, D),
                         lambda i: (jnp.minimum(blk0 + i, blk0 + nblk - 1),
                                    0)),
            pl.BlockSpec((D, CAT), lambda i: (0, 0)),
        ],
        out_specs=pl.BlockSpec((BM, CAT), lambda i: (i, 0)),
        out_shape=jax.ShapeDtypeStruct((nrows + BM, CAT), jnp.float32),
    )(x, w_cat)


def _tree_sum(vals):
    vals = list(vals)
    while len(vals) > 1:
        vals = [vals[i] + vals[i + 1] for i in range(0, len(vals) - 1, 2)] \
            + ([vals[-1]] if len(vals) % 2 else [])
    return vals[0]


_mesh = plsc.VectorSubcoreMesh(core_axis_name="c", subcore_axis_name="s")


def _make_sc_pool(nphrase, nprev):
    ppw = nphrase // NW
    nchunk = ppw // CHUNK
    prev_pw = nprev // NW if nprev else 0

    scratch = [
        pltpu.VMEM((ppw * LMAX,), jnp.int32),
        pltpu.VMEM((LMAX * C,), jnp.float32),
        pltpu.VMEM((2, RPC, CAT), jnp.float32),
        pltpu.VMEM((2, CHUNK, C), jnp.float32),
        pltpu.SemaphoreType.DMA,
        pltpu.SemaphoreType.DMA,
        pltpu.SemaphoreType.DMA,
        pltpu.SemaphoreType.DMA,
    ]
    if nprev:
        scratch.append(pltpu.VMEM((prev_pw, C), jnp.float32))
        scratch.append(pltpu.SemaphoreType.DMA)

    @functools.partial(
        pl.kernel,
        mesh=_mesh,
        out_type=jax.ShapeDtypeStruct((nprev + nphrase, C), jnp.float32),
        scratch_types=scratch,
    )
    def _sc_pool(*refs):
        if nprev:
            (cat_hbm, idx_hbm, bpos_hbm, prev_hbm, out_hbm,
             idx_v, bpos_v, rows_v, out_v,
             gsem0, gsem1, osem0, osem1, prev_v, psem) = refs
        else:
            (cat_hbm, idx_hbm, bpos_hbm, out_hbm,
             idx_v, bpos_v, rows_v, out_v,
             gsem0, gsem1, osem0, osem1) = refs
        wid = lax.axis_index("s") * NC + lax.axis_index("c")
        start_w = wid * ppw
        ostart_w = nprev + start_w
        gsem = (gsem0, gsem1)
        osem = (osem0, osem1)

        pltpu.sync_copy(idx_hbm.at[pl.ds(start_w * LMAX, ppw * LMAX)], idx_v)

        def _gather(ci, buf):
            idx_slice = idx_v.at[pl.ds(ci * RPC, RPC)]
            return pltpu.async_copy(cat_hbm.at[idx_slice], rows_v.at[buf],
                                    gsem[buf])

        _gather(0, 0)
        pltpu.sync_copy(bpos_hbm, bpos_v)
        if nprev:
            pcopy = pltpu.async_copy(
                prev_hbm.at[pl.ds(wid * prev_pw, prev_pw)], prev_v, psem)

        bpos = [[bpos_v[pl.ds(l * C + 16 * j, 16)] for j in range(NJ)]
                for l in range(LMAX)]

        def pair_body(g, carry):
            for bu in range(2):
                ci = 2 * g + bu
                nci = jnp.minimum(ci + 1, nchunk - 1)
                _gather(nci, 1 - bu)
                pltpu.make_async_copy(
                    cat_hbm.at[idx_v.at[pl.ds(ci * RPC, RPC)]],
                    rows_v.at[bu], gsem[bu]).wait()
                ob = bu

                @pl.when(ci >= 2)
                def _():
                    pltpu.make_async_copy(
                        out_v.at[ob],
                        out_hbm.at[pl.ds(ostart_w + (ci - 2) * CHUNK,
                                         CHUNK)],
                        osem[ob]).wait()

                @plsc.parallel_loop(0, CHUNK, unroll=4)
                def phrase_body(p):
                    base = p * LMAX
                    for j in range(NJ):
                        e = [jnp.exp(rows_v[bu, base + l,
                                            pl.ds(C + 16 * j, 16)]
                                     + bpos[l][j])
                             for l in range(LMAX)]
                        s = _tree_sum(e)
                        prods = [e[l] * rows_v[bu, base + l,
                                               pl.ds(16 * j, 16)]
                                 for l in range(LMAX)]
                        out_v[ob, p, pl.ds(16 * j, 16)] = _tree_sum(prods) / s

                pltpu.async_copy(
                    out_v.at[ob],
                    out_hbm.at[pl.ds(ostart_w + ci * CHUNK, CHUNK)],
                    osem[ob])
            return carry

        lax.fori_loop(0, nchunk // 2, pair_body, 0)

        if nprev:
            pcopy.wait()
            pltpu.sync_copy(prev_v, out_hbm.at[pl.ds(wid * prev_pw,
                                                     prev_pw)])
        pltpu.make_async_copy(
            cat_hbm.at[idx_v.at[pl.ds((nchunk - 1) * RPC, RPC)]],
            rows_v.at[0], gsem[0]).wait()
        for bu in range(2):
            ci = nchunk - 2 + bu
            pltpu.make_async_copy(
                out_v.at[bu],
                out_hbm.at[pl.ds(ostart_w + ci * CHUNK, CHUNK)],
                osem[bu]).wait()

    return _sc_pool


_sc_pool_first = _make_sc_pool(NB0 * P, 0)
_sc_pool_last = _make_sc_pool((B - NB0) * P, NB0 * P)


def kernel(h, phrase_mask, phrase_token_idx, W_kv, W_z, B_pos):
    w_cat = jnp.concatenate([W_kv, W_z], axis=1)
    bpos_flat = B_pos.astype(jnp.float32).reshape(-1)

    x = h.reshape(B * T, D)
    out = None
    for stage, (b0, nb) in enumerate(((0, NB0), (NB0, B - NB0))):
        sl = slice(b0, b0 + nb)
        cat = _project_rows(x, w_cat, b0 * T, nb * T)
        tok = phrase_token_idx[sl].astype(jnp.int32)
        boff = (jnp.arange(nb, dtype=jnp.int32) * T)[:, None, None]
        idx_h = jnp.where(phrase_mask[sl], tok + boff,
                          nb * T + (tok & (BM - 1))).reshape(-1)
        if stage == 0:
            out = _sc_pool_first(cat, idx_h, bpos_flat)
        else:
            out = _sc_pool_last(cat, idx_h, bpos_flat, out)
    return out.reshape(B, P, C)
